# Initial kernel scaffold; baseline (speedup 1.0000x reference)
import jax
import jax.numpy as jnp
from jax.experimental import pallas as pl


def kernel(x, edge_index, batch, W1, b1, wr1, wo1, bs1, W2, b2, wr2, wo2, bs2):
    raise NotImplementedError("write your pallas kernel here")


# speedup vs baseline: 6.8419x; 6.8419x over previous
import functools

import jax
import jax.numpy as jnp
from jax import lax
from jax.experimental import pallas as pl
from jax.experimental.pallas import tpu as pltpu
from jax.experimental.pallas import tpu_sc as plsc

_N = 10000
_E = 160000
_G = 16
_NP = 10240
_EP = 163840
_CH = 128
_NB = 5
_BM = _NP // _NB
_ROWS_T = _NP // 16
_BF = jnp.bfloat16


def _iota16():
    return lax.broadcasted_iota(jnp.int32, (16,), 0)


def _mesh():
    return plsc.VectorSubcoreMesh(core_axis_name="c", subcore_axis_name="s")


_SC_PARAMS = dict(needs_layout_passes=False, use_tc_tiling_on_sc=False)


@functools.cache
def _sc_scalar_kernel():
    return functools.partial(
        pl.kernel,
        mesh=_mesh(),
        compiler_params=pltpu.CompilerParams(**_SC_PARAMS),
        out_type=jax.ShapeDtypeStruct((2 * _NP,), jnp.float32),
        scratch_types=[
            pltpu.VMEM((_NP,), jnp.float32),
            pltpu.VMEM((_NP,), jnp.float32),
            pltpu.VMEM((_CH,), jnp.int32),
            pltpu.VMEM((_CH,), jnp.int32),
            pltpu.VMEM((_CH, 16), jnp.float32),
            pltpu.VMEM((_ROWS_T, 16), jnp.float32),
            pltpu.VMEM((_ROWS_T,), jnp.float32),
            pltpu.VMEM_SHARED((_NP, 16), jnp.float32),
        ],
    )(_sc_scalar_body)


def _sc_scalar(*args):
    return _sc_scalar_kernel()(*args)


def _sc_scalar_body(a_hbm, b_hbm, src_hbm, dst_hbm, zrow_hbm, out_hbm,
                    a_v, b_v, sidx, didx, rowbuf, cbuf, obuf, acc):
    c = lax.axis_index("c")
    s = lax.axis_index("s")
    tid = c * 16 + s
    pltpu.sync_copy(a_hbm, a_v)
    pltpu.sync_copy(b_hbm, b_v)
    pltpu.sync_copy(zrow_hbm.at[pl.ds(0, _CH)], rowbuf)
    for j in range(_ROWS_T // _CH):
        pltpu.sync_copy(rowbuf, acc.at[pl.ds(s * _ROWS_T + j * _CH, _CH)])
    plsc.subcore_barrier()

    it16 = _iota16()
    zi16 = jnp.zeros((16,), jnp.int32)
    n_chunks = _EP // 32 // _CH

    def body(i, carry):
        base = tid * (_EP // 32) + i * _CH
        pltpu.sync_copy(src_hbm.at[pl.ds(base, _CH)], sidx)
        pltpu.sync_copy(dst_hbm.at[pl.ds(base, _CH)], didx)
        for g in range(_CH // 16):
            si = sidx[pl.ds(g * 16, 16)]
            di = didx[pl.ds(g * 16, 16)]
            va = plsc.load_gather(a_v, [si])
            vb = plsc.load_gather(b_v, [di])
            plsc.store_scatter(rowbuf, [it16 + g * 16, zi16], va * vb)
        pltpu.sync_copy(rowbuf, acc.at[didx], add=True)
        return carry

    lax.fori_loop(0, n_chunks, body, 0)
    plsc.subcore_barrier()

    pltpu.sync_copy(acc.at[pl.ds(s * _ROWS_T, _ROWS_T)], cbuf)
    for j in range(_ROWS_T // 16):
        v = plsc.load_gather(cbuf, [it16 + j * 16, zi16])
        obuf[pl.ds(j * 16, 16)] = v
    pltpu.sync_copy(obuf, out_hbm.at[pl.ds(c * _NP + s * _ROWS_T, _ROWS_T)])


@functools.cache
def _sc_wide_kernel(interleaved):
    return functools.partial(
        pl.kernel,
        mesh=_mesh(),
        compiler_params=pltpu.CompilerParams(**_SC_PARAMS),
        out_type=jax.ShapeDtypeStruct((2 * _NP, 128), jnp.float32),
        scratch_types=[
            pltpu.VMEM((_CH,), jnp.int32),
            pltpu.VMEM((_CH,), jnp.int32),
            pltpu.VMEM((_CH, 128), jnp.float32),
            pltpu.SemaphoreType.DMA,
            pltpu.VMEM_SHARED((_NP, 128), jnp.float32),
        ],
    )(functools.partial(_sc_wide_body, interleaved))


def _sc_wide(tab, src, dst, zrow, interleaved=False):
    return _sc_wide_kernel(interleaved)(tab, src, dst, zrow)


def _sc_wide_body(interleaved, tab_hbm, src_hbm, dst_hbm, zrow_hbm, out_hbm,
                  sidx, didx, rows, sem, acc):
    c = lax.axis_index("c")
    s = lax.axis_index("s")
    pltpu.sync_copy(zrow_hbm.at[pl.ds(0, _CH)], rows)
    for j in range(_ROWS_T // _CH):
        pltpu.sync_copy(rows, acc.at[pl.ds(s * _ROWS_T + j * _CH, _CH)])
    plsc.subcore_barrier()

    n_chunks = _EP // 16 // _CH
    off = c * _NP

    def body(i, carry):
        base = s * (_EP // 16) + i * _CH
        pltpu.sync_copy(src_hbm.at[pl.ds(base, _CH)], sidx)
        pltpu.sync_copy(dst_hbm.at[pl.ds(base, _CH)], didx)
        for g in range(_CH // 16):
            sl = pl.ds(g * 16, 16)
            if interleaved:
                sidx[sl] = sidx[sl] * 2 + c
            else:
                sidx[sl] = sidx[sl] + off
        pltpu.async_copy(tab_hbm.at[sidx], rows, sem).wait()
        pltpu.sync_copy(rows, acc.at[didx], add=True)
        return carry

    lax.fori_loop(0, n_chunks, body, 0)
    plsc.subcore_barrier()
    for j in range(_ROWS_T // _CH):
        pltpu.sync_copy(acc.at[pl.ds(s * _ROWS_T + j * _CH, _CH)], rows)
        pltpu.sync_copy(rows,
                        out_hbm.at[pl.ds(c * _NP + s * _ROWS_T + j * _CH, _CH)])


def _mm_body(first, x_ref, w_ref, d_ref, k_ref, s_ref, o_ref):
    d = d_ref[...]
    if first:
        dinv = lax.rsqrt(d + 1.0)
        xs = x_ref[...]
    else:
        dfull = d + k_ref[...]
        dinv = jnp.where(dfull > 0.0, lax.rsqrt(jnp.maximum(dfull, 1e-30)), 0.0)
        xs = x_ref[...] * (s_ref[...] * k_ref[...])
    h = jnp.dot(xs.astype(_BF), w_ref[...].astype(_BF),
                preferred_element_type=jnp.float32)
    o_ref[0, ...] = h * dinv


def _mm(x, w, dcol, kcol=None, scol=None, first=True):
    ione = jnp.ones((_NP, 1), jnp.float32)
    kcol = ione if kcol is None else kcol
    scol = ione if scol is None else scol
    return pl.pallas_call(
        functools.partial(_mm_body, first),
        grid=(2, _NB),
        in_specs=[
            pl.BlockSpec((_BM, 256), lambda c, i: (i, 0)),
            pl.BlockSpec((256, 128), lambda c, i: (0, c)),
            pl.BlockSpec((_BM, 1), lambda c, i: (i, 0)),
            pl.BlockSpec((_BM, 1), lambda c, i: (i, 0)),
            pl.BlockSpec((_BM, 1), lambda c, i: (i, 0)),
        ],
        out_specs=pl.BlockSpec((1, _BM, 128), lambda c, i: (c, i, 0)),
        out_shape=jax.ShapeDtypeStruct((2, _NP, 128), jnp.float32),
    )(x, w, dcol, kcol, scol)


def _ep_body(first, a0_ref, a1_ref, t0_ref, t1_ref, d_ref, k_ref, b_ref,
             h_ref):
    d = d_ref[...]
    if first:
        dinv = lax.rsqrt(d + 1.0)
    else:
        dfull = d + k_ref[...]
        dinv = jnp.where(dfull > 0.0, lax.rsqrt(jnp.maximum(dfull, 1e-30)), 0.0)
    b = b_ref[...]
    h0 = jnp.maximum(dinv * (a0_ref[...] + t0_ref[...]) + b[:, :128], 0.0)
    h1 = jnp.maximum(dinv * (a1_ref[...] + t1_ref[...]) + b[:, 128:], 0.0)
    h = jnp.concatenate([h0, h1], axis=1)
    if not first:
        h = h * k_ref[...]
    h_ref[...] = h


def _ep(acc, tab, dcol, kcol, brow, first=True):
    if kcol is None:
        kcol = jnp.ones((_NP, 1), jnp.float32)
    a0, a1 = acc[:_NP], acc[_NP:]
    t0, t1 = tab[0], tab[1]
    return pl.pallas_call(
        functools.partial(_ep_body, first),
        grid=(_NB,),
        in_specs=[
            pl.BlockSpec((_BM, 128), lambda i: (i, 0)),
            pl.BlockSpec((_BM, 128), lambda i: (i, 0)),
            pl.BlockSpec((_BM, 128), lambda i: (i, 0)),
            pl.BlockSpec((_BM, 128), lambda i: (i, 0)),
            pl.BlockSpec((_BM, 1), lambda i: (i, 0)),
            pl.BlockSpec((_BM, 1), lambda i: (i, 0)),
            pl.BlockSpec((1, 256), lambda i: (0, 0)),
        ],
        out_specs=pl.BlockSpec((_BM, 256), lambda i: (i, 0)),
        out_shape=jax.ShapeDtypeStruct((_NP, 256), jnp.float32),
    )(a0, a1, t0, t1, dcol, kcol, brow)


def _score_body(a0_ref, a1_ref, h_ref, ww_ref, bs_ref, z_ref):
    agg = jnp.concatenate([a0_ref[...], a1_ref[...]], axis=1)
    ww = ww_ref[...]
    p = jnp.dot(agg.astype(_BF), ww[:256].astype(_BF),
                preferred_element_type=jnp.float32)
    q = jnp.dot(h_ref[...].astype(_BF), ww[256:].astype(_BF),
                preferred_element_type=jnp.float32)
    z_ref[...] = (p[:, :1] + q[:, :1]) + bs_ref[0, 0]


def _score(agg, h, ww, bs):
    a0, a1 = agg[:_NP], agg[_NP:]
    return pl.pallas_call(
        _score_body,
        grid=(_NB,),
        in_specs=[
            pl.BlockSpec((_BM, 128), lambda i: (i, 0)),
            pl.BlockSpec((_BM, 128), lambda i: (i, 0)),
            pl.BlockSpec((_BM, 256), lambda i: (i, 0)),
            pl.BlockSpec((512, 128), lambda i: (0, 0)),
            pl.BlockSpec((1, 1), lambda i: (0, 0)),
        ],
        out_specs=pl.BlockSpec((_BM, 1), lambda i: (i, 0)),
        out_shape=jax.ShapeDtypeStruct((_NP, 1), jnp.float32),
    )(a0, a1, h, ww, bs)


def _topk_body(z_ref, bm_ref, vin_ref, keep_ref, s_ref):
    s = jnp.tanh(z_ref[...])
    s_ref[...] = s
    bits = lax.bitcast_convert_type(s, jnp.int32)
    imin = jnp.int32(-2147483648)
    key = jnp.where(bits >= 0, bits, imin - bits) + jnp.int32(0x40000000)
    bm = bm_ref[...]
    valid = (vin_ref[...] > 0.0) & (bm < _G)
    r0 = lax.broadcasted_iota(jnp.int32, (80, 128), 0)
    r1 = lax.broadcasted_iota(jnp.int32, (80, 128), 1)
    idxm = r0 * 128 + r1
    keep = jnp.zeros((80, 128), jnp.bool_)
    one = jnp.int32(1)
    for g in range(_G):
        mg = valid & (bm == g)
        mgf = jnp.where(mg, 1.0, 0.0)
        cntg = jnp.sum(mgf)
        kg = jnp.ceil(0.5 * cntg)

        def bbody(b, t):
            tt = t | (one << (30 - b))
            cb = jnp.sum(jnp.where(mg & (key >= tt), 1.0, 0.0))
            return jnp.where(cb >= kg, tt, t)

        tau = lax.fori_loop(0, 31, bbody, jnp.int32(0))
        cgt = jnp.sum(jnp.where(mg & (key > tau), 1.0, 0.0))
        r = kg - cgt
        eq = mg & (key == tau)

        def ibody(b, t):
            tt = t + (one << (13 - b))
            cb = jnp.sum(jnp.where(eq & (idxm < tt), 1.0, 0.0))
            return jnp.where(cb < r, tt, t)

        iot = lax.fori_loop(0, 14, ibody, jnp.int32(0))
        keep = keep | (mg & ((key > tau) | (eq & (idxm <= iot))))
    keep_ref[...] = jnp.where(keep, 1.0, 0.0)


def _topk(z2d, bm2, vin):
    return pl.pallas_call(
        _topk_body,
        grid=(1,),
        in_specs=[pl.BlockSpec((80, 128), lambda i: (0, 0))] * 3,
        out_specs=[pl.BlockSpec((80, 128), lambda i: (0, 0))] * 2,
        out_shape=[jax.ShapeDtypeStruct((80, 128), jnp.float32)] * 2,
    )(z2d, bm2, vin)


def _final_body(bm_ref, w_ref, k_ref, h2_ref, o_ref, acc_s, cnt_s):
    i = pl.program_id(0)

    @pl.when(i == 0)
    def _():
        acc_s[...] = jnp.zeros((_G, 256), jnp.float32)
        cnt_s[...] = jnp.zeros((_G, 1), jnp.float32)

    g16 = lax.broadcasted_iota(jnp.int32, (_G, _BM), 0)
    oh = bm_ref[...] == g16
    ohw = jnp.where(oh, w_ref[...], 0.0)
    acc_s[...] += jnp.dot(ohw, h2_ref[...], preferred_element_type=jnp.float32,
                          precision=lax.Precision.HIGHEST)
    ohk = jnp.where(oh, k_ref[...], 0.0)
    cnt_s[...] += jnp.sum(ohk, axis=1, keepdims=True)

    @pl.when(i == _NB - 1)
    def _():
        o_ref[...] = acc_s[...] / jnp.maximum(cnt_s[...], 1.0)


def _final(bm13, w13, k13, h2):
    return pl.pallas_call(
        _final_body,
        grid=(_NB,),
        in_specs=[
            pl.BlockSpec((1, _BM), lambda i: (0, i)),
            pl.BlockSpec((1, _BM), lambda i: (0, i)),
            pl.BlockSpec((1, _BM), lambda i: (0, i)),
            pl.BlockSpec((_BM, 256), lambda i: (i, 0)),
        ],
        out_specs=pl.BlockSpec((_G, 256), lambda i: (0, 0)),
        out_shape=jax.ShapeDtypeStruct((_G, 256), jnp.float32),
        scratch_shapes=[
            pltpu.VMEM((_G, 256), jnp.float32),
            pltpu.VMEM((_G, 1), jnp.float32),
        ],
    )(bm13, w13, k13, h2)


def kernel(x, edge_index, batch, W1, b1, wr1, wo1, bs1, W2, b2, wr2, wo2, bs2):
    f32 = jnp.float32
    xp = jnp.zeros((_NP, 256), f32).at[:_N].set(x)
    srcp = jnp.full((_EP,), _N, jnp.int32).at[:_E].set(edge_index[0])
    dstp = jnp.full((_EP,), _N, jnp.int32).at[:_E].set(edge_index[1])
    bmp = jnp.full((_NP,), _G, jnp.int32).at[:_N].set(batch)
    bm2 = bmp.reshape(80, 128)
    ones = jnp.ones((_NP,), f32)
    zrow = jnp.zeros((_CH, 128), f32)
    zrow16 = jnp.zeros((_CH, 16), f32)
    ww1 = jnp.zeros((512, 128), f32).at[:256, 0].set(wr1[:, 0]).at[256:, 0].set(wo1[:, 0])
    ww2 = jnp.zeros((512, 128), f32).at[:256, 0].set(wr2[:, 0]).at[256:, 0].set(wo2[:, 0])
    b1r = b1.reshape(1, 256)
    b2r = b2.reshape(1, 256)
    bs1r = bs1.reshape(1, 1)
    bs2r = bs2.reshape(1, 1)

    degp = _sc_scalar(ones, ones, srcp, dstp, zrow16)
    d1 = (degp[:_NP] + degp[_NP:]).reshape(_NP, 1)
    t1 = _mm(xp, W1, d1, first=True)
    acc1 = _sc_wide(t1.reshape(2 * _NP, 128), srcp, dstp, zrow)
    h = _ep(acc1, t1, d1, None, b1r, first=True)
    agg1 = _sc_wide(h.reshape(2 * _NP, 128), srcp, dstp, zrow, interleaved=True)
    z1 = _score(agg1, h, ww1, bs1r)
    keep1, s1 = _topk(z1.reshape(80, 128), bm2, jnp.ones((80, 128), f32))
    k1c = keep1.reshape(_NP, 1)
    s1c = s1.reshape(_NP, 1)

    degp2 = _sc_scalar(k1c[:, 0], k1c[:, 0], srcp, dstp, zrow16)
    d2 = (degp2[:_NP] + degp2[_NP:]).reshape(_NP, 1)
    t2 = _mm(h, W2, d2, kcol=k1c, scol=s1c, first=False)
    acc2 = _sc_wide(t2.reshape(2 * _NP, 128), srcp, dstp, zrow)
    h2k = _ep(acc2, t2, d2, k1c, b2r, first=False)
    agg2 = _sc_wide(h2k.reshape(2 * _NP, 128), srcp, dstp, zrow, interleaved=True)
    z2 = _score(agg2, h2k, ww2, bs2r)
    keep2, s2 = _topk(z2.reshape(80, 128), bm2, keep1)

    w13 = (s2 * keep2).reshape(1, _NP)
    k13 = keep2.reshape(1, _NP)
    bm13 = bmp.reshape(1, _NP)
    return _final(bm13, w13, k13, h2k)


# speedup vs baseline: 8.0863x; 1.1819x over previous
import functools

import jax
import jax.numpy as jnp
from jax import lax
from jax.experimental import pallas as pl
from jax.experimental.pallas import tpu as pltpu
from jax.experimental.pallas import tpu_sc as plsc

_N = 10000
_E = 160000
_G = 16
_NP = 10240
_EP = 163840
_CH = 128
_NB = 5
_BM = _NP // _NB
_ROWS_T = _NP // 16
_BF = jnp.bfloat16


def _iota16():
    return lax.broadcasted_iota(jnp.int32, (16,), 0)


def _mesh():
    return plsc.VectorSubcoreMesh(core_axis_name="c", subcore_axis_name="s")


_SC_PARAMS = dict(needs_layout_passes=False, use_tc_tiling_on_sc=False)


@functools.cache
def _sc_scalar_kernel():
    return functools.partial(
        pl.kernel,
        mesh=_mesh(),
        compiler_params=pltpu.CompilerParams(**_SC_PARAMS),
        out_type=jax.ShapeDtypeStruct((2 * _NP,), jnp.float32),
        scratch_types=[
            pltpu.VMEM((_NP,), jnp.float32),
            pltpu.VMEM((_NP,), jnp.float32),
            pltpu.VMEM((_CH,), jnp.int32),
            pltpu.VMEM((_CH,), jnp.int32),
            pltpu.VMEM((_CH, 16), jnp.float32),
            pltpu.VMEM((_ROWS_T, 16), jnp.float32),
            pltpu.VMEM((_ROWS_T,), jnp.float32),
            pltpu.VMEM_SHARED((_NP, 16), jnp.float32),
        ],
    )(_sc_scalar_body)


def _sc_scalar(*args):
    return _sc_scalar_kernel()(*args)


def _sc_scalar_body(a_hbm, b_hbm, src_hbm, dst_hbm, zrow_hbm, out_hbm,
                    a_v, b_v, sidx, didx, rowbuf, cbuf, obuf, acc):
    c = lax.axis_index("c")
    s = lax.axis_index("s")
    tid = c * 16 + s
    pltpu.sync_copy(a_hbm, a_v)
    pltpu.sync_copy(b_hbm, b_v)
    pltpu.sync_copy(zrow_hbm.at[pl.ds(0, _CH)], rowbuf)
    for j in range(_ROWS_T // _CH):
        pltpu.sync_copy(rowbuf, acc.at[pl.ds(s * _ROWS_T + j * _CH, _CH)])
    plsc.subcore_barrier()

    it16 = _iota16()
    zi16 = jnp.zeros((16,), jnp.int32)
    n_chunks = _EP // 32 // _CH

    def body(i, carry):
        base = tid * (_EP // 32) + i * _CH
        pltpu.sync_copy(src_hbm.at[pl.ds(base, _CH)], sidx)
        pltpu.sync_copy(dst_hbm.at[pl.ds(base, _CH)], didx)
        for g in range(_CH // 16):
            si = sidx[pl.ds(g * 16, 16)]
            di = didx[pl.ds(g * 16, 16)]
            va = plsc.load_gather(a_v, [si])
            vb = plsc.load_gather(b_v, [di])
            plsc.store_scatter(rowbuf, [it16 + g * 16, zi16], va * vb)
        pltpu.sync_copy(rowbuf, acc.at[didx], add=True)
        return carry

    lax.fori_loop(0, n_chunks, body, 0)
    plsc.subcore_barrier()

    pltpu.sync_copy(acc.at[pl.ds(s * _ROWS_T, _ROWS_T)], cbuf)
    for j in range(_ROWS_T // 16):
        v = plsc.load_gather(cbuf, [it16 + j * 16, zi16])
        obuf[pl.ds(j * 16, 16)] = v
    pltpu.sync_copy(obuf, out_hbm.at[pl.ds(c * _NP + s * _ROWS_T, _ROWS_T)])


_NBUF = 2
_CPT = _EP // 16 // _CH


@functools.cache
def _sc_wide_kernel(interleaved):
    return functools.partial(
        pl.kernel,
        mesh=_mesh(),
        compiler_params=pltpu.CompilerParams(**_SC_PARAMS),
        out_type=jax.ShapeDtypeStruct((2 * _NP, 128), jnp.float32),
        scratch_types=[
            pltpu.VMEM((_CPT // 2, _CH), jnp.int32),
            pltpu.VMEM((_CPT // 2, _CH), jnp.int32),
            [pltpu.VMEM((_CH, 128), jnp.float32) for _ in range(_NBUF)],
            pltpu.SemaphoreType.DMA,
            pltpu.SemaphoreType.DMA,
            pltpu.VMEM_SHARED((_NP, 128), jnp.float32),
        ],
    )(functools.partial(_sc_wide_body, interleaved))


def _sc_wide(tab, src, dst, zrow, interleaved=False):
    return _sc_wide_kernel(interleaved)(tab, src, dst, zrow)


def _sc_wide_body(interleaved, tab_hbm, src_hbm, dst_hbm, zrow_hbm, out_hbm,
                  sidx, didx, rowsb, gsem, ssem, acc):
    c = lax.axis_index("c")
    s = lax.axis_index("s")
    hp = _CPT // 2
    pltpu.sync_copy(zrow_hbm.at[pl.ds(0, _CH)], rowsb[0])
    for j in range(_ROWS_T // _CH):
        pltpu.sync_copy(rowsb[0], acc.at[pl.ds(s * _ROWS_T + j * _CH, _CH)])
    plsc.subcore_barrier()

    off = c * _NP

    def tbody(i, carry):
        for g in range(_CH // 16):
            sl = pl.ds(g * 16, 16)
            if interleaved:
                sidx[i, sl] = sidx[i, sl] * 2 + c
            else:
                sidx[i, sl] = sidx[i, sl] + off
        return carry

    def body(j, carry):
        i = j * _NBUF
        gs = [pltpu.async_copy(tab_hbm.at[sidx.at[i + u]], rowsb[u], gsem)
              for u in range(_NBUF)]
        for g in gs:
            g.wait()
        ss = [pltpu.async_copy(rowsb[u], acc.at[didx.at[i + u]], ssem,
                               add=True)
              for u in range(_NBUF)]
        for t in ss:
            t.wait()
        return carry

    for p in range(2):
        base = s * _CPT + p * hp
        pltpu.sync_copy(src_hbm.at[pl.ds(base, hp)], sidx)
        pltpu.sync_copy(dst_hbm.at[pl.ds(base, hp)], didx)
        lax.fori_loop(0, hp, tbody, 0)
        lax.fori_loop(0, hp // _NBUF, body, 0)
    plsc.subcore_barrier()
    for j in range(_ROWS_T // _CH):
        pltpu.sync_copy(acc.at[pl.ds(s * _ROWS_T + j * _CH, _CH)], rowsb[0])
        pltpu.sync_copy(rowsb[0],
                        out_hbm.at[pl.ds(c * _NP + s * _ROWS_T + j * _CH, _CH)])


def _mm_body(first, x_ref, w_ref, d_ref, k_ref, s_ref, o_ref):
    d = d_ref[...]
    if first:
        dinv = lax.rsqrt(d + 1.0)
        xs = x_ref[...]
    else:
        dfull = d + k_ref[...]
        dinv = jnp.where(dfull > 0.0, lax.rsqrt(jnp.maximum(dfull, 1e-30)), 0.0)
        xs = x_ref[...] * (s_ref[...] * k_ref[...])
    h = jnp.dot(xs.astype(_BF), w_ref[...].astype(_BF),
                preferred_element_type=jnp.float32)
    o_ref[0, ...] = h * dinv


def _mm(x, w, dcol, kcol=None, scol=None, first=True):
    ione = jnp.ones((_NP, 1), jnp.float32)
    kcol = ione if kcol is None else kcol
    scol = ione if scol is None else scol
    return pl.pallas_call(
        functools.partial(_mm_body, first),
        grid=(2, _NB),
        in_specs=[
            pl.BlockSpec((_BM, 256), lambda c, i: (i, 0)),
            pl.BlockSpec((256, 128), lambda c, i: (0, c)),
            pl.BlockSpec((_BM, 1), lambda c, i: (i, 0)),
            pl.BlockSpec((_BM, 1), lambda c, i: (i, 0)),
            pl.BlockSpec((_BM, 1), lambda c, i: (i, 0)),
        ],
        out_specs=pl.BlockSpec((1, _BM, 128), lambda c, i: (c, i, 0)),
        out_shape=jax.ShapeDtypeStruct((2, _NP, 128), jnp.float32),
    )(x, w, dcol, kcol, scol)


def _ep_body(first, a0_ref, a1_ref, t0_ref, t1_ref, d_ref, k_ref, b_ref,
             h_ref):
    d = d_ref[...]
    if first:
        dinv = lax.rsqrt(d + 1.0)
    else:
        dfull = d + k_ref[...]
        dinv = jnp.where(dfull > 0.0, lax.rsqrt(jnp.maximum(dfull, 1e-30)), 0.0)
    b = b_ref[...]
    h0 = jnp.maximum(dinv * (a0_ref[...] + t0_ref[...]) + b[:, :128], 0.0)
    h1 = jnp.maximum(dinv * (a1_ref[...] + t1_ref[...]) + b[:, 128:], 0.0)
    h = jnp.concatenate([h0, h1], axis=1)
    if not first:
        h = h * k_ref[...]
    h_ref[...] = h


def _ep(acc, tab, dcol, kcol, brow, first=True):
    if kcol is None:
        kcol = jnp.ones((_NP, 1), jnp.float32)
    a0, a1 = acc[:_NP], acc[_NP:]
    t0, t1 = tab[0], tab[1]
    return pl.pallas_call(
        functools.partial(_ep_body, first),
        grid=(_NB,),
        in_specs=[
            pl.BlockSpec((_BM, 128), lambda i: (i, 0)),
            pl.BlockSpec((_BM, 128), lambda i: (i, 0)),
            pl.BlockSpec((_BM, 128), lambda i: (i, 0)),
            pl.BlockSpec((_BM, 128), lambda i: (i, 0)),
            pl.BlockSpec((_BM, 1), lambda i: (i, 0)),
            pl.BlockSpec((_BM, 1), lambda i: (i, 0)),
            pl.BlockSpec((1, 256), lambda i: (0, 0)),
        ],
        out_specs=pl.BlockSpec((_BM, 256), lambda i: (i, 0)),
        out_shape=jax.ShapeDtypeStruct((_NP, 256), jnp.float32),
    )(a0, a1, t0, t1, dcol, kcol, brow)


def _score_body(a0_ref, a1_ref, h_ref, ww_ref, bs_ref, z_ref):
    agg = jnp.concatenate([a0_ref[...], a1_ref[...]], axis=1)
    ww = ww_ref[...]
    p = jnp.dot(agg.astype(_BF), ww[:256].astype(_BF),
                preferred_element_type=jnp.float32)
    q = jnp.dot(h_ref[...].astype(_BF), ww[256:].astype(_BF),
                preferred_element_type=jnp.float32)
    z_ref[...] = (p[:, :1] + q[:, :1]) + bs_ref[0, 0]


def _score(agg, h, ww, bs):
    a0, a1 = agg[:_NP], agg[_NP:]
    return pl.pallas_call(
        _score_body,
        grid=(_NB,),
        in_specs=[
            pl.BlockSpec((_BM, 128), lambda i: (i, 0)),
            pl.BlockSpec((_BM, 128), lambda i: (i, 0)),
            pl.BlockSpec((_BM, 256), lambda i: (i, 0)),
            pl.BlockSpec((512, 128), lambda i: (0, 0)),
            pl.BlockSpec((1, 1), lambda i: (0, 0)),
        ],
        out_specs=pl.BlockSpec((_BM, 1), lambda i: (i, 0)),
        out_shape=jax.ShapeDtypeStruct((_NP, 1), jnp.float32),
    )(a0, a1, h, ww, bs)


def _topk_body(z_ref, bm_ref, vin_ref, keep_ref, s_ref):
    s = jnp.tanh(z_ref[...])
    s_ref[...] = s
    bits = lax.bitcast_convert_type(s, jnp.int32)
    imin = jnp.int32(-2147483648)
    key = jnp.where(bits >= 0, bits, imin - bits) + jnp.int32(0x40000000)
    bm = bm_ref[...]
    valid = (vin_ref[...] > 0.0) & (bm < _G)
    r0 = lax.broadcasted_iota(jnp.int32, (80, 128), 0)
    r1 = lax.broadcasted_iota(jnp.int32, (80, 128), 1)
    idxm = r0 * 128 + r1
    keep = jnp.zeros((80, 128), jnp.bool_)
    one = jnp.int32(1)
    for g in range(_G):
        mg = valid & (bm == g)
        mgf = jnp.where(mg, 1.0, 0.0)
        cntg = jnp.sum(mgf)
        kg = jnp.ceil(0.5 * cntg)

        def bbody(b, t):
            tt = t | (one << (30 - b))
            cb = jnp.sum(jnp.where(mg & (key >= tt), 1.0, 0.0))
            return jnp.where(cb >= kg, tt, t)

        tau = lax.fori_loop(0, 31, bbody, jnp.int32(0))
        cgt = jnp.sum(jnp.where(mg & (key > tau), 1.0, 0.0))
        r = kg - cgt
        eq = mg & (key == tau)

        def ibody(b, t):
            tt = t + (one << (13 - b))
            cb = jnp.sum(jnp.where(eq & (idxm < tt), 1.0, 0.0))
            return jnp.where(cb < r, tt, t)

        iot = lax.fori_loop(0, 14, ibody, jnp.int32(0))
        keep = keep | (mg & ((key > tau) | (eq & (idxm <= iot))))
    keep_ref[...] = jnp.where(keep, 1.0, 0.0)


def _topk(z2d, bm2, vin):
    return pl.pallas_call(
        _topk_body,
        grid=(1,),
        in_specs=[pl.BlockSpec((80, 128), lambda i: (0, 0))] * 3,
        out_specs=[pl.BlockSpec((80, 128), lambda i: (0, 0))] * 2,
        out_shape=[jax.ShapeDtypeStruct((80, 128), jnp.float32)] * 2,
    )(z2d, bm2, vin)


def _final_body(bm_ref, w_ref, k_ref, h2_ref, o_ref, acc_s, cnt_s):
    i = pl.program_id(0)

    @pl.when(i == 0)
    def _():
        acc_s[...] = jnp.zeros((_G, 256), jnp.float32)
        cnt_s[...] = jnp.zeros((_G, 1), jnp.float32)

    g16 = lax.broadcasted_iota(jnp.int32, (_G, _BM), 0)
    oh = bm_ref[...] == g16
    ohw = jnp.where(oh, w_ref[...], 0.0)
    acc_s[...] += jnp.dot(ohw, h2_ref[...], preferred_element_type=jnp.float32,
                          precision=lax.Precision.HIGHEST)
    ohk = jnp.where(oh, k_ref[...], 0.0)
    cnt_s[...] += jnp.sum(ohk, axis=1, keepdims=True)

    @pl.when(i == _NB - 1)
    def _():
        o_ref[...] = acc_s[...] / jnp.maximum(cnt_s[...], 1.0)


def _final(bm13, w13, k13, h2):
    return pl.pallas_call(
        _final_body,
        grid=(_NB,),
        in_specs=[
            pl.BlockSpec((1, _BM), lambda i: (0, i)),
            pl.BlockSpec((1, _BM), lambda i: (0, i)),
            pl.BlockSpec((1, _BM), lambda i: (0, i)),
            pl.BlockSpec((_BM, 256), lambda i: (i, 0)),
        ],
        out_specs=pl.BlockSpec((_G, 256), lambda i: (0, 0)),
        out_shape=jax.ShapeDtypeStruct((_G, 256), jnp.float32),
        scratch_shapes=[
            pltpu.VMEM((_G, 256), jnp.float32),
            pltpu.VMEM((_G, 1), jnp.float32),
        ],
    )(bm13, w13, k13, h2)


def kernel(x, edge_index, batch, W1, b1, wr1, wo1, bs1, W2, b2, wr2, wo2, bs2):
    f32 = jnp.float32
    xp = jnp.zeros((_NP, 256), f32).at[:_N].set(x)
    srcp = jnp.full((_EP,), _N, jnp.int32).at[:_E].set(edge_index[0])
    dstp = jnp.full((_EP,), _N, jnp.int32).at[:_E].set(edge_index[1])
    bmp = jnp.full((_NP,), _G, jnp.int32).at[:_N].set(batch)
    bm2 = bmp.reshape(80, 128)
    srcp2 = srcp.reshape(_EP // _CH, _CH)
    dstp2 = dstp.reshape(_EP // _CH, _CH)
    ones = jnp.ones((_NP,), f32)
    zrow = jnp.zeros((_CH, 128), f32)
    zrow16 = jnp.zeros((_CH, 16), f32)
    ww1 = jnp.zeros((512, 128), f32).at[:256, 0].set(wr1[:, 0]).at[256:, 0].set(wo1[:, 0])
    ww2 = jnp.zeros((512, 128), f32).at[:256, 0].set(wr2[:, 0]).at[256:, 0].set(wo2[:, 0])
    b1r = b1.reshape(1, 256)
    b2r = b2.reshape(1, 256)
    bs1r = bs1.reshape(1, 1)
    bs2r = bs2.reshape(1, 1)

    degp = _sc_scalar(ones, ones, srcp, dstp, zrow16)
    d1 = (degp[:_NP] + degp[_NP:]).reshape(_NP, 1)
    t1 = _mm(xp, W1, d1, first=True)
    acc1 = _sc_wide(t1.reshape(2 * _NP, 128), srcp2, dstp2, zrow)
    h = _ep(acc1, t1, d1, None, b1r, first=True)
    agg1 = _sc_wide(h.reshape(2 * _NP, 128), srcp2, dstp2, zrow, interleaved=True)
    z1 = _score(agg1, h, ww1, bs1r)
    keep1, s1 = _topk(z1.reshape(80, 128), bm2, jnp.ones((80, 128), f32))
    k1c = keep1.reshape(_NP, 1)
    s1c = s1.reshape(_NP, 1)

    degp2 = _sc_scalar(k1c[:, 0], k1c[:, 0], srcp, dstp, zrow16)
    d2 = (degp2[:_NP] + degp2[_NP:]).reshape(_NP, 1)
    t2 = _mm(h, W2, d2, kcol=k1c, scol=s1c, first=False)
    acc2 = _sc_wide(t2.reshape(2 * _NP, 128), srcp2, dstp2, zrow)
    h2k = _ep(acc2, t2, d2, k1c, b2r, first=False)
    agg2 = _sc_wide(h2k.reshape(2 * _NP, 128), srcp2, dstp2, zrow, interleaved=True)
    z2 = _score(agg2, h2k, ww2, bs2r)
    keep2, s2 = _topk(z2.reshape(80, 128), bm2, keep1)

    w13 = (s2 * keep2).reshape(1, _NP)
    k13 = keep2.reshape(1, _NP)
    bm13 = bmp.reshape(1, _NP)
    return _final(bm13, w13, k13, h2k)


# speedup vs baseline: 8.1161x; 1.0037x over previous
import functools

import jax
import jax.numpy as jnp
from jax import lax
from jax.experimental import pallas as pl
from jax.experimental.pallas import tpu as pltpu
from jax.experimental.pallas import tpu_sc as plsc

_N = 10000
_E = 160000
_G = 16
_NP = 10240
_EP = 163840
_CH = 128
_NB = 5
_BM = _NP // _NB
_ROWS_T = _NP // 16
_BF = jnp.bfloat16


def _iota16():
    return lax.broadcasted_iota(jnp.int32, (16,), 0)


def _mesh():
    return plsc.VectorSubcoreMesh(core_axis_name="c", subcore_axis_name="s")


_SC_PARAMS = dict(needs_layout_passes=False, use_tc_tiling_on_sc=False)


@functools.cache
def _sc_scalar_kernel():
    return functools.partial(
        pl.kernel,
        mesh=_mesh(),
        compiler_params=pltpu.CompilerParams(**_SC_PARAMS),
        out_type=jax.ShapeDtypeStruct((2 * _NP,), jnp.float32),
        scratch_types=[
            pltpu.VMEM((_NP,), jnp.float32),
            pltpu.VMEM((_NP,), jnp.float32),
            pltpu.VMEM((_CH,), jnp.int32),
            pltpu.VMEM((_CH,), jnp.int32),
            pltpu.VMEM((_CH, 16), jnp.float32),
            pltpu.VMEM((_ROWS_T, 16), jnp.float32),
            pltpu.VMEM((_ROWS_T,), jnp.float32),
            pltpu.VMEM_SHARED((_NP, 16), jnp.float32),
        ],
    )(_sc_scalar_body)


def _sc_scalar(*args):
    return _sc_scalar_kernel()(*args)


def _sc_scalar_body(a_hbm, b_hbm, src_hbm, dst_hbm, zrow_hbm, out_hbm,
                    a_v, b_v, sidx, didx, rowbuf, cbuf, obuf, acc):
    c = lax.axis_index("c")
    s = lax.axis_index("s")
    tid = c * 16 + s
    pltpu.sync_copy(a_hbm, a_v)
    pltpu.sync_copy(b_hbm, b_v)
    pltpu.sync_copy(zrow_hbm.at[pl.ds(0, _CH)], rowbuf)
    for j in range(_ROWS_T // _CH):
        pltpu.sync_copy(rowbuf, acc.at[pl.ds(s * _ROWS_T + j * _CH, _CH)])
    plsc.subcore_barrier()

    it16 = _iota16()
    zi16 = jnp.zeros((16,), jnp.int32)
    n_chunks = _EP // 32 // _CH

    def body(i, carry):
        base = tid * (_EP // 32) + i * _CH
        pltpu.sync_copy(src_hbm.at[pl.ds(base, _CH)], sidx)
        pltpu.sync_copy(dst_hbm.at[pl.ds(base, _CH)], didx)
        for g in range(_CH // 16):
            si = sidx[pl.ds(g * 16, 16)]
            di = didx[pl.ds(g * 16, 16)]
            va = plsc.load_gather(a_v, [si])
            vb = plsc.load_gather(b_v, [di])
            plsc.store_scatter(rowbuf, [it16 + g * 16, zi16], va * vb)
        pltpu.sync_copy(rowbuf, acc.at[didx], add=True)
        return carry

    lax.fori_loop(0, n_chunks, body, 0)
    plsc.subcore_barrier()

    pltpu.sync_copy(acc.at[pl.ds(s * _ROWS_T, _ROWS_T)], cbuf)
    for j in range(_ROWS_T // 16):
        v = plsc.load_gather(cbuf, [it16 + j * 16, zi16])
        obuf[pl.ds(j * 16, 16)] = v
    pltpu.sync_copy(obuf, out_hbm.at[pl.ds(c * _NP + s * _ROWS_T, _ROWS_T)])


_NBUF = 4
_CHW = 80
_CPT = _EP // 16 // _CHW
_PH = 4


@functools.cache
def _sc_wide_kernel(interleaved):
    return functools.partial(
        pl.kernel,
        mesh=_mesh(),
        compiler_params=pltpu.CompilerParams(**_SC_PARAMS),
        out_type=jax.ShapeDtypeStruct((2 * _NP, 128), jnp.float32),
        scratch_types=[
            pltpu.VMEM((_CPT // _PH, _CHW), jnp.int32),
            pltpu.VMEM((_CPT // _PH, _CHW), jnp.int32),
            [pltpu.VMEM((_CHW, 128), jnp.float32) for _ in range(_NBUF)],
            pltpu.SemaphoreType.DMA,
            pltpu.SemaphoreType.DMA,
            pltpu.VMEM_SHARED((_NP, 128), jnp.float32),
        ],
    )(functools.partial(_sc_wide_body, interleaved))


def _sc_wide(tab, src, dst, zrow, interleaved=False):
    return _sc_wide_kernel(interleaved)(tab, src, dst, zrow)


def _sc_wide_body(interleaved, tab_hbm, src_hbm, dst_hbm, zrow_hbm, out_hbm,
                  sidx, didx, rowsb, gsem, ssem, acc):
    c = lax.axis_index("c")
    s = lax.axis_index("s")
    hp = _CPT // _PH
    pltpu.sync_copy(zrow_hbm.at[pl.ds(0, _CHW)], rowsb[0])
    for j in range(_ROWS_T // _CHW):
        pltpu.sync_copy(rowsb[0], acc.at[pl.ds(s * _ROWS_T + j * _CHW, _CHW)])
    plsc.subcore_barrier()

    off = c * _NP

    def tbody(i, carry):
        for g in range(_CHW // 16):
            sl = pl.ds(g * 16, 16)
            if interleaved:
                sidx[i, sl] = sidx[i, sl] * 2 + c
            else:
                sidx[i, sl] = sidx[i, sl] + off
        return carry

    def body(j, carry):
        i = j * _NBUF
        gs = [pltpu.async_copy(tab_hbm.at[sidx.at[i + u]], rowsb[u], gsem)
              for u in range(_NBUF)]
        for g in gs:
            g.wait()
        ss = [pltpu.async_copy(rowsb[u], acc.at[didx.at[i + u]], ssem,
                               add=True)
              for u in range(_NBUF)]
        for t in ss:
            t.wait()
        return carry

    for p in range(_PH):
        base = s * _CPT + p * hp
        pltpu.sync_copy(src_hbm.at[pl.ds(base, hp)], sidx)
        pltpu.sync_copy(dst_hbm.at[pl.ds(base, hp)], didx)
        lax.fori_loop(0, hp, tbody, 0)
        lax.fori_loop(0, hp // _NBUF, body, 0)
    plsc.subcore_barrier()
    for j in range(_ROWS_T // _CHW):
        pltpu.sync_copy(acc.at[pl.ds(s * _ROWS_T + j * _CHW, _CHW)], rowsb[0])
        pltpu.sync_copy(rowsb[0],
                        out_hbm.at[pl.ds(c * _NP + s * _ROWS_T + j * _CHW, _CHW)])


def _mm_body(first, x_ref, w_ref, d_ref, k_ref, s_ref, o_ref):
    d = d_ref[...]
    if first:
        dinv = lax.rsqrt(d + 1.0)
        xs = x_ref[...]
    else:
        dfull = d + k_ref[...]
        dinv = jnp.where(dfull > 0.0, lax.rsqrt(jnp.maximum(dfull, 1e-30)), 0.0)
        xs = x_ref[...] * (s_ref[...] * k_ref[...])
    h = jnp.dot(xs.astype(_BF), w_ref[...].astype(_BF),
                preferred_element_type=jnp.float32)
    o_ref[0, ...] = h * dinv


def _mm(x, w, dcol, kcol=None, scol=None, first=True):
    ione = jnp.ones((_NP, 1), jnp.float32)
    kcol = ione if kcol is None else kcol
    scol = ione if scol is None else scol
    return pl.pallas_call(
        functools.partial(_mm_body, first),
        grid=(2, _NB),
        in_specs=[
            pl.BlockSpec((_BM, 256), lambda c, i: (i, 0)),
            pl.BlockSpec((256, 128), lambda c, i: (0, c)),
            pl.BlockSpec((_BM, 1), lambda c, i: (i, 0)),
            pl.BlockSpec((_BM, 1), lambda c, i: (i, 0)),
            pl.BlockSpec((_BM, 1), lambda c, i: (i, 0)),
        ],
        out_specs=pl.BlockSpec((1, _BM, 128), lambda c, i: (c, i, 0)),
        out_shape=jax.ShapeDtypeStruct((2, _NP, 128), jnp.float32),
    )(x, w, dcol, kcol, scol)


def _ep_body(first, a0_ref, a1_ref, t0_ref, t1_ref, d_ref, k_ref, b_ref,
             h_ref):
    d = d_ref[...]
    if first:
        dinv = lax.rsqrt(d + 1.0)
    else:
        dfull = d + k_ref[...]
        dinv = jnp.where(dfull > 0.0, lax.rsqrt(jnp.maximum(dfull, 1e-30)), 0.0)
    b = b_ref[...]
    h0 = jnp.maximum(dinv * (a0_ref[...] + t0_ref[...]) + b[:, :128], 0.0)
    h1 = jnp.maximum(dinv * (a1_ref[...] + t1_ref[...]) + b[:, 128:], 0.0)
    h = jnp.concatenate([h0, h1], axis=1)
    if not first:
        h = h * k_ref[...]
    h_ref[...] = h


def _ep(acc, tab, dcol, kcol, brow, first=True):
    if kcol is None:
        kcol = jnp.ones((_NP, 1), jnp.float32)
    a0, a1 = acc[:_NP], acc[_NP:]
    t0, t1 = tab[0], tab[1]
    return pl.pallas_call(
        functools.partial(_ep_body, first),
        grid=(_NB,),
        in_specs=[
            pl.BlockSpec((_BM, 128), lambda i: (i, 0)),
            pl.BlockSpec((_BM, 128), lambda i: (i, 0)),
            pl.BlockSpec((_BM, 128), lambda i: (i, 0)),
            pl.BlockSpec((_BM, 128), lambda i: (i, 0)),
            pl.BlockSpec((_BM, 1), lambda i: (i, 0)),
            pl.BlockSpec((_BM, 1), lambda i: (i, 0)),
            pl.BlockSpec((1, 256), lambda i: (0, 0)),
        ],
        out_specs=pl.BlockSpec((_BM, 256), lambda i: (i, 0)),
        out_shape=jax.ShapeDtypeStruct((_NP, 256), jnp.float32),
    )(a0, a1, t0, t1, dcol, kcol, brow)


def _score_body(a0_ref, a1_ref, h_ref, ww_ref, bs_ref, z_ref):
    agg = jnp.concatenate([a0_ref[...], a1_ref[...]], axis=1)
    ww = ww_ref[...]
    p = jnp.dot(agg.astype(_BF), ww[:256].astype(_BF),
                preferred_element_type=jnp.float32)
    q = jnp.dot(h_ref[...].astype(_BF), ww[256:].astype(_BF),
                preferred_element_type=jnp.float32)
    z_ref[...] = (p[:, :1] + q[:, :1]) + bs_ref[0, 0]


def _score(agg, h, ww, bs):
    a0, a1 = agg[:_NP], agg[_NP:]
    return pl.pallas_call(
        _score_body,
        grid=(_NB,),
        in_specs=[
            pl.BlockSpec((_BM, 128), lambda i: (i, 0)),
            pl.BlockSpec((_BM, 128), lambda i: (i, 0)),
            pl.BlockSpec((_BM, 256), lambda i: (i, 0)),
            pl.BlockSpec((512, 128), lambda i: (0, 0)),
            pl.BlockSpec((1, 1), lambda i: (0, 0)),
        ],
        out_specs=pl.BlockSpec((_BM, 1), lambda i: (i, 0)),
        out_shape=jax.ShapeDtypeStruct((_NP, 1), jnp.float32),
    )(a0, a1, h, ww, bs)


def _topk_body(z_ref, bm_ref, vin_ref, keep_ref, s_ref):
    s = jnp.tanh(z_ref[...])
    s_ref[...] = s
    bits = lax.bitcast_convert_type(s, jnp.int32)
    imin = jnp.int32(-2147483648)
    key = jnp.where(bits >= 0, bits, imin - bits) + jnp.int32(0x40000000)
    bm = bm_ref[...]
    valid = (vin_ref[...] > 0.0) & (bm < _G)
    r0 = lax.broadcasted_iota(jnp.int32, (80, 128), 0)
    r1 = lax.broadcasted_iota(jnp.int32, (80, 128), 1)
    idxm = r0 * 128 + r1
    keep = jnp.zeros((80, 128), jnp.bool_)
    one = jnp.int32(1)
    for g in range(_G):
        mg = valid & (bm == g)
        mgf = jnp.where(mg, 1.0, 0.0)
        cntg = jnp.sum(mgf)
        kg = jnp.ceil(0.5 * cntg)

        def bbody(b, t):
            tt = t | (one << (30 - b))
            cb = jnp.sum(jnp.where(mg & (key >= tt), 1.0, 0.0))
            return jnp.where(cb >= kg, tt, t)

        tau = lax.fori_loop(0, 31, bbody, jnp.int32(0))
        cgt = jnp.sum(jnp.where(mg & (key > tau), 1.0, 0.0))
        r = kg - cgt
        eq = mg & (key == tau)

        def ibody(b, t):
            tt = t + (one << (13 - b))
            cb = jnp.sum(jnp.where(eq & (idxm < tt), 1.0, 0.0))
            return jnp.where(cb < r, tt, t)

        iot = lax.fori_loop(0, 14, ibody, jnp.int32(0))
        keep = keep | (mg & ((key > tau) | (eq & (idxm <= iot))))
    keep_ref[...] = jnp.where(keep, 1.0, 0.0)


def _topk(z2d, bm2, vin):
    return pl.pallas_call(
        _topk_body,
        grid=(1,),
        in_specs=[pl.BlockSpec((80, 128), lambda i: (0, 0))] * 3,
        out_specs=[pl.BlockSpec((80, 128), lambda i: (0, 0))] * 2,
        out_shape=[jax.ShapeDtypeStruct((80, 128), jnp.float32)] * 2,
    )(z2d, bm2, vin)


def _final_body(bm_ref, w_ref, k_ref, h2_ref, o_ref, acc_s, cnt_s):
    i = pl.program_id(0)

    @pl.when(i == 0)
    def _():
        acc_s[...] = jnp.zeros((_G, 256), jnp.float32)
        cnt_s[...] = jnp.zeros((_G, 1), jnp.float32)

    g16 = lax.broadcasted_iota(jnp.int32, (_G, _BM), 0)
    oh = bm_ref[...] == g16
    ohw = jnp.where(oh, w_ref[...], 0.0)
    acc_s[...] += jnp.dot(ohw, h2_ref[...], preferred_element_type=jnp.float32,
                          precision=lax.Precision.HIGHEST)
    ohk = jnp.where(oh, k_ref[...], 0.0)
    cnt_s[...] += jnp.sum(ohk, axis=1, keepdims=True)

    @pl.when(i == _NB - 1)
    def _():
        o_ref[...] = acc_s[...] / jnp.maximum(cnt_s[...], 1.0)


def _final(bm13, w13, k13, h2):
    return pl.pallas_call(
        _final_body,
        grid=(_NB,),
        in_specs=[
            pl.BlockSpec((1, _BM), lambda i: (0, i)),
            pl.BlockSpec((1, _BM), lambda i: (0, i)),
            pl.BlockSpec((1, _BM), lambda i: (0, i)),
            pl.BlockSpec((_BM, 256), lambda i: (i, 0)),
        ],
        out_specs=pl.BlockSpec((_G, 256), lambda i: (0, 0)),
        out_shape=jax.ShapeDtypeStruct((_G, 256), jnp.float32),
        scratch_shapes=[
            pltpu.VMEM((_G, 256), jnp.float32),
            pltpu.VMEM((_G, 1), jnp.float32),
        ],
    )(bm13, w13, k13, h2)


def kernel(x, edge_index, batch, W1, b1, wr1, wo1, bs1, W2, b2, wr2, wo2, bs2):
    f32 = jnp.float32
    xp = jnp.zeros((_NP, 256), f32).at[:_N].set(x)
    srcp = jnp.full((_EP,), _N, jnp.int32).at[:_E].set(edge_index[0])
    dstp = jnp.full((_EP,), _N, jnp.int32).at[:_E].set(edge_index[1])
    bmp = jnp.full((_NP,), _G, jnp.int32).at[:_N].set(batch)
    bm2 = bmp.reshape(80, 128)
    srcp2 = srcp.reshape(_EP // _CHW, _CHW)
    dstp2 = dstp.reshape(_EP // _CHW, _CHW)
    ones = jnp.ones((_NP,), f32)
    zrow = jnp.zeros((_CH, 128), f32)
    zrow16 = jnp.zeros((_CH, 16), f32)
    ww1 = jnp.zeros((512, 128), f32).at[:256, 0].set(wr1[:, 0]).at[256:, 0].set(wo1[:, 0])
    ww2 = jnp.zeros((512, 128), f32).at[:256, 0].set(wr2[:, 0]).at[256:, 0].set(wo2[:, 0])
    b1r = b1.reshape(1, 256)
    b2r = b2.reshape(1, 256)
    bs1r = bs1.reshape(1, 1)
    bs2r = bs2.reshape(1, 1)

    degp = _sc_scalar(ones, ones, srcp, dstp, zrow16)
    d1 = (degp[:_NP] + degp[_NP:]).reshape(_NP, 1)
    t1 = _mm(xp, W1, d1, first=True)
    acc1 = _sc_wide(t1.reshape(2 * _NP, 128), srcp2, dstp2, zrow)
    h = _ep(acc1, t1, d1, None, b1r, first=True)
    agg1 = _sc_wide(h.reshape(2 * _NP, 128), srcp2, dstp2, zrow, interleaved=True)
    z1 = _score(agg1, h, ww1, bs1r)
    keep1, s1 = _topk(z1.reshape(80, 128), bm2, jnp.ones((80, 128), f32))
    k1c = keep1.reshape(_NP, 1)
    s1c = s1.reshape(_NP, 1)

    degp2 = _sc_scalar(k1c[:, 0], k1c[:, 0], srcp, dstp, zrow16)
    d2 = (degp2[:_NP] + degp2[_NP:]).reshape(_NP, 1)
    t2 = _mm(h, W2, d2, kcol=k1c, scol=s1c, first=False)
    acc2 = _sc_wide(t2.reshape(2 * _NP, 128), srcp2, dstp2, zrow)
    h2k = _ep(acc2, t2, d2, k1c, b2r, first=False)
    agg2 = _sc_wide(h2k.reshape(2 * _NP, 128), srcp2, dstp2, zrow, interleaved=True)
    z2 = _score(agg2, h2k, ww2, bs2r)
    keep2, s2 = _topk(z2.reshape(80, 128), bm2, keep1)

    w13 = (s2 * keep2).reshape(1, _NP)
    k13 = keep2.reshape(1, _NP)
    bm13 = bmp.reshape(1, _NP)
    return _final(bm13, w13, k13, h2k)
